# Initial kernel scaffold; baseline (speedup 1.0000x reference)
import jax
import jax.numpy as jnp
from jax.experimental import pallas as pl


def kernel(x, edge_index, W1, b1, W2, b2, Wfc, bfc):
    raise NotImplementedError("write your pallas kernel here")


# speedup vs baseline: 13.0648x; 13.0648x over previous
import functools

import jax
import jax.numpy as jnp
from jax import lax
from jax.experimental import pallas as pl
from jax.experimental.pallas import tpu as pltpu
from jax.experimental.pallas import tpu_sc as plsc

N = 10000
NPAD = 10240
E = 320000
NC, NS = 2, 16
NW = NC * NS
EPW = E // NW
K = 80
RPT = NPAD // NS

f32 = jnp.float32
_MESH = functools.partial(
    plsc.VectorSubcoreMesh, core_axis_name="c", subcore_axis_name="s")


def _deg_body(dst_hbm, out_hbm, idx_v, ones_v, zbuf_v, acc_sh):
    c = lax.axis_index("c")
    s = lax.axis_index("s")
    for j in range(K // 16):
        ones_v[pl.ds(j * 16, 16)] = jnp.full((16,), 1.0, f32)
    for j in range(RPT // 16):
        zbuf_v[pl.ds(j * 16, 16)] = jnp.zeros((16,), f32)
    pltpu.sync_copy(zbuf_v, acc_sh.at[pl.ds(s * RPT, RPT)])
    plsc.subcore_barrier()
    base = (c * NS + s) * EPW

    def chunk(i, carry):
        pltpu.sync_copy(dst_hbm.at[pl.ds(base + i * K, K)], idx_v)
        pltpu.sync_copy(ones_v, acc_sh.at[idx_v], add=True)
        return carry

    lax.fori_loop(0, EPW // K, chunk, 0)
    plsc.subcore_barrier()
    pltpu.sync_copy(acc_sh.at[pl.ds(s * RPT, RPT)],
                    out_hbm.at[c].at[pl.ds(s * RPT, RPT)])


def _deg(dst):
    return pl.kernel(
        _deg_body,
        mesh=_MESH(),
        out_type=jax.ShapeDtypeStruct((NC, NPAD), f32),
        scratch_types=[
            pltpu.VMEM((K,), jnp.int32),
            pltpu.VMEM((K,), f32),
            pltpu.VMEM((RPT,), f32),
            pltpu.VMEM_SHARED((NPAD,), f32),
        ],
    )(dst)


def _make_agg(D):
    def body(tbl_hbm, src_hbm, dst_hbm, out_hbm, idx_s, idx_d, rows_v, sem,
             acc_sh):
        c = lax.axis_index("c")
        s = lax.axis_index("s")
        pltpu.sync_copy(tbl_hbm.at[pl.ds(s * RPT, RPT)],
                        acc_sh.at[pl.ds(s * RPT, RPT)])
        plsc.subcore_barrier()
        base = (c * NS + s) * EPW

        def chunk(i, carry):
            off = base + i * K
            pltpu.sync_copy(src_hbm.at[pl.ds(off, K)], idx_s)
            pltpu.sync_copy(dst_hbm.at[pl.ds(off, K)], idx_d)
            pltpu.async_copy(tbl_hbm.at[idx_s], rows_v, sem).wait()
            pltpu.sync_copy(rows_v, acc_sh.at[idx_d], add=True)
            return carry

        lax.fori_loop(0, EPW // K, chunk, 0)
        plsc.subcore_barrier()
        pltpu.sync_copy(acc_sh.at[pl.ds(s * RPT, RPT)],
                        out_hbm.at[c].at[pl.ds(s * RPT, RPT)])

    def run(tbl, src, dst):
        return pl.kernel(
            body,
            mesh=_MESH(),
            out_type=jax.ShapeDtypeStruct((NC, NPAD, D), f32),
            scratch_types=[
                pltpu.VMEM((K,), jnp.int32),
                pltpu.VMEM((K,), jnp.int32),
                pltpu.VMEM((K, D), f32),
                pltpu.SemaphoreType.DMA,
                pltpu.VMEM_SHARED((NPAD, D), f32),
            ],
        )(tbl, src, dst)

    return run


_agg128 = _make_agg(128)


def _tc_a(x, W1, degp):
    def body(x_ref, w_ref, degp_ref, tbl_ref, dis_ref):
        deg = degp_ref[0] + degp_ref[1] + 1.0
        dis = lax.rsqrt(deg)
        dis_ref[...] = dis
        h = jnp.dot(x_ref[...], w_ref[...], preferred_element_type=f32)
        tbl_ref[:N, :] = h * dis[:N]
        tbl_ref[N:, :] = jnp.zeros((NPAD - N, 128), f32)

    return pl.pallas_call(
        body,
        out_shape=(jax.ShapeDtypeStruct((NPAD, 128), f32),
                   jax.ShapeDtypeStruct((NPAD, 1), f32)),
    )(x, W1, degp)


def _tc_b(s1, tbl1, dis, b1, W2):
    def body(s1_ref, tbl_ref, dis_ref, b1_ref, w2_ref, out_ref):
        agg = s1_ref[0] + s1_ref[1] - tbl_ref[...]
        z = jnp.maximum(agg * dis_ref[...] + b1_ref[...], 0.0)
        h2 = jnp.dot(z[:N], w2_ref[...], preferred_element_type=f32)
        out_ref[:N, :64] = h2 * dis_ref[:N]
        out_ref[:N, 64:] = jnp.zeros((N, 64), f32)
        out_ref[N:, :] = jnp.zeros((NPAD - N, 128), f32)

    return pl.pallas_call(
        body,
        out_shape=jax.ShapeDtypeStruct((NPAD, 128), f32),
    )(s1, tbl1, dis, b1, W2)


def _tc_c(s2, tbl2, dis, b2, Wfc, bfc):
    def body(s2_ref, tbl_ref, dis_ref, b2_ref, wfc_ref, bfc_ref, out_ref):
        agg = (s2_ref[0] + s2_ref[1] - tbl_ref[...])[:, :64]
        z = jnp.maximum(agg * dis_ref[...] + b2_ref[...], 0.0)
        out_ref[...] = (jnp.dot(z[:N], wfc_ref[...],
                                preferred_element_type=f32) + bfc_ref[...])

    return pl.pallas_call(
        body,
        out_shape=jax.ShapeDtypeStruct((N, 112), f32),
    )(s2, tbl2, dis, b2, Wfc, bfc)


def kernel(x, edge_index, W1, b1, W2, b2, Wfc, bfc):
    ei = edge_index.astype(jnp.int32)
    src, dst = ei[0], ei[1]
    degp = _deg(dst)
    tbl1, dis = _tc_a(x, W1, degp.reshape(NC, NPAD, 1))
    s1 = _agg128(tbl1, src, dst)
    tbl2 = _tc_b(s1, tbl1, dis, b1, W2)
    s2 = _agg128(tbl2, src, dst)
    return _tc_c(s2, tbl2, dis, b2, Wfc, bfc)


# speedup vs baseline: 26.4390x; 2.0237x over previous
import functools

import jax
import jax.numpy as jnp
from jax import lax
from jax.experimental import pallas as pl
from jax.experimental.pallas import tpu as pltpu
from jax.experimental.pallas import tpu_sc as plsc

N = 10000
NPAD = 10240
E = 320000
NC, NS = 2, 16
NW = NC * NS
EPW = E // NW
K = 80
RPT = NPAD // NS

f32 = jnp.float32
_MESH = functools.partial(
    plsc.VectorSubcoreMesh, core_axis_name="c", subcore_axis_name="s")


NCHUNK = EPW // K
NBUF = 4
NGRP = NCHUNK // NBUF


DNBUF = 5
DNGRP = NCHUNK // DNBUF


def _deg_body(dst_hbm, out_hbm, ones_v, zbuf_v, acc_sh, *bufs):
    idx_v = bufs[:DNBUF]
    sem_i = bufs[DNBUF:2 * DNBUF]
    sem_s = bufs[2 * DNBUF:3 * DNBUF]
    c = lax.axis_index("c")
    s = lax.axis_index("s")
    for j in range(K // 16):
        ones_v[pl.ds(j * 16, 16)] = jnp.full((16,), 1.0, f32)
    for j in range(RPT // 16):
        zbuf_v[pl.ds(j * 16, 16)] = jnp.zeros((16,), f32)
    pltpu.sync_copy(zbuf_v, acc_sh.at[pl.ds(s * RPT, RPT)])
    plsc.subcore_barrier()
    base = (c * NS + s) * EPW

    for b in range(DNBUF):
        pltpu.async_copy(dst_hbm.at[pl.ds(base + b * K, K)], idx_v[b],
                         sem_i[b])

    def group(g, carry):
        scs = []
        for b in range(DNBUF):
            off = base + (g * DNBUF + b) * K
            pltpu.make_async_copy(dst_hbm.at[pl.ds(off, K)], idx_v[b],
                                  sem_i[b]).wait()
            scs.append(pltpu.async_copy(ones_v, acc_sh.at[idx_v[b]],
                                        sem_s[b], add=True))
        for b in range(DNBUF):
            scs[b].wait()

            @pl.when(g < DNGRP - 1)
            def _():
                off2 = base + ((g + 1) * DNBUF + b) * K
                pltpu.async_copy(dst_hbm.at[pl.ds(off2, K)], idx_v[b],
                                 sem_i[b])
        return carry

    lax.fori_loop(0, DNGRP, group, 0)
    plsc.subcore_barrier()
    pltpu.sync_copy(acc_sh.at[pl.ds(s * RPT, RPT)],
                    out_hbm.at[c].at[pl.ds(s * RPT, RPT)])


def _deg(dst):
    return pl.kernel(
        _deg_body,
        mesh=_MESH(),
        out_type=jax.ShapeDtypeStruct((NC, NPAD), f32),
        scratch_types=(
            [pltpu.VMEM((K,), f32),
             pltpu.VMEM((RPT,), f32),
             pltpu.VMEM_SHARED((NPAD,), f32)]
            + [pltpu.VMEM((K,), jnp.int32) for _ in range(DNBUF)]
            + [pltpu.SemaphoreType.DMA for _ in range(2 * DNBUF)]
        ),
    )(dst)


def _make_agg(D):
    def body(tbl_hbm, src_hbm, dst_hbm, out_hbm, acc_sh, *bufs):
        idx_s = bufs[0 * NBUF:1 * NBUF]
        idx_d = bufs[1 * NBUF:2 * NBUF]
        rows = bufs[2 * NBUF:3 * NBUF]
        sem_is = bufs[3 * NBUF:4 * NBUF]
        sem_id = bufs[4 * NBUF:5 * NBUF]
        sem_g = bufs[5 * NBUF:6 * NBUF]
        sem_sc = bufs[6 * NBUF:7 * NBUF]
        c = lax.axis_index("c")
        s = lax.axis_index("s")
        base = (c * NS + s) * EPW
        for b in range(NBUF):
            pltpu.async_copy(src_hbm.at[pl.ds(base + b * K, K)], idx_s[b],
                             sem_is[b])
            pltpu.async_copy(dst_hbm.at[pl.ds(base + b * K, K)], idx_d[b],
                             sem_id[b])
        pltpu.sync_copy(tbl_hbm.at[pl.ds(s * RPT, RPT)],
                        acc_sh.at[pl.ds(s * RPT, RPT)])
        plsc.subcore_barrier()

        def group(g, carry):
            gs, scs = [], []
            for b in range(NBUF):
                off = base + (g * NBUF + b) * K
                pltpu.make_async_copy(src_hbm.at[pl.ds(off, K)], idx_s[b],
                                      sem_is[b]).wait()
                gs.append(pltpu.async_copy(tbl_hbm.at[idx_s[b]], rows[b],
                                           sem_g[b]))
            for b in range(NBUF):
                off = base + (g * NBUF + b) * K
                pltpu.make_async_copy(dst_hbm.at[pl.ds(off, K)], idx_d[b],
                                      sem_id[b]).wait()
                gs[b].wait()
                scs.append(pltpu.async_copy(rows[b], acc_sh.at[idx_d[b]],
                                            sem_sc[b], add=True))
                @pl.when(g < (NGRP if b == 0 else NGRP - 1))
                def _():
                    off2 = base + ((g + 1) * NBUF + b) * K
                    pltpu.async_copy(src_hbm.at[pl.ds(off2, K)], idx_s[b],
                                     sem_is[b])
            for b in range(NBUF):
                scs[b].wait()

                @pl.when(g < (NGRP if b == 0 else NGRP - 1))
                def _():
                    off2 = base + ((g + 1) * NBUF + b) * K
                    pltpu.async_copy(dst_hbm.at[pl.ds(off2, K)], idx_d[b],
                                     sem_id[b])
            return carry

        lax.fori_loop(0, NGRP, group, 0)
        off = base + (NCHUNK - 1) * K
        pltpu.make_async_copy(src_hbm.at[pl.ds(off, K)], idx_s[0],
                              sem_is[0]).wait()
        pltpu.async_copy(tbl_hbm.at[idx_s[0]], rows[0], sem_g[0]).wait()
        pltpu.make_async_copy(dst_hbm.at[pl.ds(off, K)], idx_d[0],
                              sem_id[0]).wait()
        pltpu.async_copy(rows[0], acc_sh.at[idx_d[0]], sem_sc[0],
                         add=True).wait()
        plsc.subcore_barrier()
        pltpu.sync_copy(acc_sh.at[pl.ds(s * RPT, RPT)],
                        out_hbm.at[c].at[pl.ds(s * RPT, RPT)])

    def run(tbl, src, dst):
        return pl.kernel(
            body,
            mesh=_MESH(),
            out_type=jax.ShapeDtypeStruct((NC, NPAD, D), f32),
            scratch_types=(
                [pltpu.VMEM_SHARED((NPAD, D), f32)]
                + [pltpu.VMEM((K,), jnp.int32) for _ in range(2 * NBUF)]
                + [pltpu.VMEM((K, D), f32) for _ in range(NBUF)]
                + [pltpu.SemaphoreType.DMA for _ in range(4 * NBUF)]
            ),
        )(tbl, src, dst)

    return run


_agg128 = _make_agg(128)


def _tc_a(x, W1, degp):
    def body(x_ref, w_ref, degp_ref, tbl_ref, dis_ref):
        deg = degp_ref[0] + degp_ref[1] + 1.0
        dis = lax.rsqrt(deg)
        dis_ref[...] = dis
        h = jnp.dot(x_ref[...], w_ref[...], preferred_element_type=f32)
        tbl_ref[:N, :] = h * dis[:N]
        tbl_ref[N:, :] = jnp.zeros((NPAD - N, 128), f32)

    return pl.pallas_call(
        body,
        out_shape=(jax.ShapeDtypeStruct((NPAD, 128), f32),
                   jax.ShapeDtypeStruct((NPAD, 1), f32)),
    )(x, W1, degp)


def _tc_b(s1, tbl1, dis, b1, W2):
    def body(s1_ref, tbl_ref, dis_ref, b1_ref, w2_ref, out_ref):
        agg = s1_ref[0] + s1_ref[1] - tbl_ref[...]
        z = jnp.maximum(agg * dis_ref[...] + b1_ref[...], 0.0)
        h2 = jnp.dot(z[:N], w2_ref[...], preferred_element_type=f32)
        out_ref[:N, :64] = h2 * dis_ref[:N]
        out_ref[:N, 64:] = jnp.zeros((N, 64), f32)
        out_ref[N:, :] = jnp.zeros((NPAD - N, 128), f32)

    return pl.pallas_call(
        body,
        out_shape=jax.ShapeDtypeStruct((NPAD, 128), f32),
    )(s1, tbl1, dis, b1, W2)


def _tc_c(s2, tbl2, dis, b2, Wfc, bfc):
    def body(s2_ref, tbl_ref, dis_ref, b2_ref, wfc_ref, bfc_ref, out_ref):
        agg = (s2_ref[0] + s2_ref[1] - tbl_ref[...])[:, :64]
        z = jnp.maximum(agg * dis_ref[...] + b2_ref[...], 0.0)
        out_ref[...] = (jnp.dot(z[:N], wfc_ref[...],
                                preferred_element_type=f32) + bfc_ref[...])

    return pl.pallas_call(
        body,
        out_shape=jax.ShapeDtypeStruct((N, 112), f32),
    )(s2, tbl2, dis, b2, Wfc, bfc)


def kernel(x, edge_index, W1, b1, W2, b2, Wfc, bfc):
    ei = edge_index.astype(jnp.int32)
    src, dst = ei[0], ei[1]
    degp = _deg(dst)
    tbl1, dis = _tc_a(x, W1, degp.reshape(NC, NPAD, 1))
    s1 = _agg128(tbl1, src, dst)
    tbl2 = _tc_b(s1, tbl1, dis, b1, W2)
    s2 = _agg128(tbl2, src, dst)
    return _tc_c(s2, tbl2, dis, b2, Wfc, bfc)


# speedup vs baseline: 34.2855x; 1.2968x over previous
import functools

import jax
import jax.numpy as jnp
from jax import lax
from jax.experimental import pallas as pl
from jax.experimental.pallas import tpu as pltpu
from jax.experimental.pallas import tpu_sc as plsc

N = 10000
NPAD = 10240
E = 320000
NC, NS = 2, 16
NW = NC * NS
EPW = E // NW
K = 80
RPT = NPAD // NS

f32 = jnp.float32
_MESH = functools.partial(
    plsc.VectorSubcoreMesh, core_axis_name="c", subcore_axis_name="s")


NCHUNK = EPW // K


DNBUF = 5
DNGRP = NCHUNK // DNBUF


def _deg_body(dst_hbm, out_hbm, ones_v, zbuf_v, acc_sh, *bufs):
    idx_v = bufs[:DNBUF]
    sem_i = bufs[DNBUF:2 * DNBUF]
    sem_s = bufs[2 * DNBUF:3 * DNBUF]
    c = lax.axis_index("c")
    s = lax.axis_index("s")
    for j in range(K // 16):
        ones_v[pl.ds(j * 16, 16)] = jnp.full((16,), 1.0, f32)
    for j in range(RPT // 16):
        zbuf_v[pl.ds(j * 16, 16)] = jnp.zeros((16,), f32)
    pltpu.sync_copy(zbuf_v, acc_sh.at[pl.ds(s * RPT, RPT)])
    plsc.subcore_barrier()
    base = (c * NS + s) * EPW

    for b in range(DNBUF):
        pltpu.async_copy(dst_hbm.at[pl.ds(base + b * K, K)], idx_v[b],
                         sem_i[b])

    def group(g, carry):
        scs = []
        for b in range(DNBUF):
            off = base + (g * DNBUF + b) * K
            pltpu.make_async_copy(dst_hbm.at[pl.ds(off, K)], idx_v[b],
                                  sem_i[b]).wait()
            scs.append(pltpu.async_copy(ones_v, acc_sh.at[idx_v[b]],
                                        sem_s[b], add=True))
        for b in range(DNBUF):
            scs[b].wait()

            @pl.when(g < DNGRP - 1)
            def _():
                off2 = base + ((g + 1) * DNBUF + b) * K
                pltpu.async_copy(dst_hbm.at[pl.ds(off2, K)], idx_v[b],
                                 sem_i[b])
        return carry

    lax.fori_loop(0, DNGRP, group, 0)
    plsc.subcore_barrier()
    pltpu.sync_copy(acc_sh.at[pl.ds(s * RPT, RPT)],
                    out_hbm.at[c].at[pl.ds(s * RPT, RPT)])


def _deg(dst):
    return pl.kernel(
        _deg_body,
        mesh=_MESH(),
        out_type=jax.ShapeDtypeStruct((NC, NPAD), f32),
        scratch_types=(
            [pltpu.VMEM((K,), f32),
             pltpu.VMEM((RPT,), f32),
             pltpu.VMEM_SHARED((NPAD,), f32)]
            + [pltpu.VMEM((K,), jnp.int32) for _ in range(DNBUF)]
            + [pltpu.SemaphoreType.DMA for _ in range(2 * DNBUF)]
        ),
    )(dst)


AK = 40
ANC = EPW // AK
RBUF = 5
IBUF = 2 * RBUF
ANIT = ANC // IBUF


def _make_agg(D, tc_tiling):
    def body(tbl_hbm, src_hbm, dst_hbm, out_hbm, acc_sh, *bufs):
        idx_s = bufs[0 * IBUF:1 * IBUF]
        idx_d = bufs[1 * IBUF:2 * IBUF]
        rows = bufs[2 * IBUF:2 * IBUF + RBUF]
        o = 2 * IBUF + RBUF
        sem_is = bufs[o:o + IBUF]
        sem_id = bufs[o + IBUF:o + 2 * IBUF]
        sem_g = bufs[o + 2 * IBUF:o + 2 * IBUF + RBUF]
        sem_sc = bufs[o + 2 * IBUF + RBUF:o + 2 * IBUF + 2 * RBUF]
        c = lax.axis_index("c")
        s = lax.axis_index("s")
        base = (c * NS + s) * EPW

        def start_idx(i, slot):
            off = base + i * AK
            pltpu.async_copy(src_hbm.at[pl.ds(off, AK)], idx_s[slot],
                             sem_is[slot])
            pltpu.async_copy(dst_hbm.at[pl.ds(off, AK)], idx_d[slot],
                             sem_id[slot])

        def wait_is(i, slot):
            pltpu.make_async_copy(src_hbm.at[pl.ds(base + i * AK, AK)],
                                  idx_s[slot], sem_is[slot]).wait()

        def wait_id(i, slot):
            pltpu.make_async_copy(dst_hbm.at[pl.ds(base + i * AK, AK)],
                                  idx_d[slot], sem_id[slot]).wait()

        def start_gather(slot, r):
            return pltpu.async_copy(tbl_hbm.at[idx_s[slot]], rows[r],
                                    sem_g[r])

        def wait_gather(slot, r):
            pltpu.make_async_copy(tbl_hbm.at[idx_s[slot]], rows[r],
                                  sem_g[r]).wait()

        def start_scatter(slot, r):
            return pltpu.async_copy(rows[r], acc_sh.at[idx_d[slot]],
                                    sem_sc[r], add=True)

        def wait_scatter(slot, r):
            pltpu.make_async_copy(rows[r], acc_sh.at[idx_d[slot]],
                                  sem_sc[r]).wait()

        for j in range(RBUF):
            start_idx(j, j)
        pltpu.sync_copy(tbl_hbm.at[pl.ds(s * RPT, RPT)],
                        acc_sh.at[pl.ds(s * RPT, RPT)])
        plsc.subcore_barrier()

        def iteration(t, carry):
            i0 = t * IBUF
            for j in range(IBUF):
                i = i0 + j
                r = j % RBUF
                jm2, rm2 = (j - 2) % IBUF, (j - 2) % RBUF
                jm5 = (j - RBUF) % IBUF

                def step1():
                    wait_scatter(jm5, r)

                if j >= RBUF:
                    step1()
                else:
                    pl.when(t > 0)(step1)
                wait_is(i, j)
                start_gather(j, r)

                def step3():
                    wait_gather(jm2, rm2)
                    wait_id(i - 2, jm2)
                    start_scatter(jm2, rm2)

                if j >= 2:
                    step3()
                else:
                    pl.when(t > 0)(step3)

                def step4():
                    start_idx(i + RBUF, (j + RBUF) % IBUF)

                if j < RBUF:
                    step4()
                else:
                    pl.when(t < ANIT - 1)(step4)
            return carry

        lax.fori_loop(0, ANIT, iteration, 0)
        last = ANC - 2
        for i in range(last, ANC):
            j, r = i % IBUF, i % RBUF
            wait_gather(j, r)
            wait_id(i, j)
            start_scatter(j, r)
        for i in range(ANC - RBUF, ANC):
            wait_scatter(i % IBUF, i % RBUF)
        plsc.subcore_barrier()
        pltpu.sync_copy(acc_sh.at[pl.ds(s * RPT, RPT)],
                        out_hbm.at[c].at[pl.ds(s * RPT, RPT)])

    def run(tbl, src, dst):
        return pl.kernel(
            body,
            mesh=_MESH(),
            out_type=jax.ShapeDtypeStruct((NC, NPAD, D), f32),
            scratch_types=(
                [pltpu.VMEM_SHARED((NPAD, D), f32)]
                + [pltpu.VMEM((AK,), jnp.int32) for _ in range(2 * IBUF)]
                + [pltpu.VMEM((AK, D), f32) for _ in range(RBUF)]
                + [pltpu.SemaphoreType.DMA for _ in range(2 * IBUF + 2 * RBUF)]
            ),
            compiler_params=pltpu.CompilerParams(
                use_tc_tiling_on_sc=tc_tiling),
        )(tbl, src, dst)

    return run


_agg128 = _make_agg(128, True)
_agg64 = _make_agg(64, False)


def _tc_mm(x, W1):
    def body(x_ref, w_ref, h_ref):
        h_ref[...] = jnp.dot(x_ref[...], w_ref[...],
                             preferred_element_type=f32)

    return pl.pallas_call(
        body,
        out_shape=jax.ShapeDtypeStruct((N, 128), f32),
    )(x, W1)


def _tc_a(h, degp):
    def body(h_ref, degp_ref, tbl_ref, dis_ref):
        deg = degp_ref[0] + degp_ref[1] + 1.0
        dis = lax.rsqrt(deg)
        dis_ref[...] = dis
        tbl_ref[:N, :] = h_ref[...] * dis[:N]
        tbl_ref[N:, :] = jnp.zeros((NPAD - N, 128), f32)

    return pl.pallas_call(
        body,
        out_shape=(jax.ShapeDtypeStruct((NPAD, 128), f32),
                   jax.ShapeDtypeStruct((NPAD, 1), f32)),
    )(h, degp)


def _tc_b(s1, tbl1, dis, b1, W2):
    def body(s1_ref, tbl_ref, dis_ref, b1_ref, w2_ref, out_ref):
        agg = s1_ref[0] + s1_ref[1] - tbl_ref[...]
        z = jnp.maximum(agg * dis_ref[...] + b1_ref[...], 0.0)
        h2 = jnp.dot(z[:N], w2_ref[...], preferred_element_type=f32)
        out_ref[:N, :] = h2 * dis_ref[:N]
        out_ref[N:, :] = jnp.zeros((NPAD - N, 64), f32)

    return pl.pallas_call(
        body,
        out_shape=jax.ShapeDtypeStruct((NPAD, 64), f32),
    )(s1, tbl1, dis, b1, W2)


def _tc_c(s2, tbl2, dis, b2, Wfc, bfc):
    def body(s2_ref, tbl_ref, dis_ref, b2_ref, wfc_ref, bfc_ref, out_ref):
        agg = s2_ref[0] + s2_ref[1] - tbl_ref[...]
        z = jnp.maximum(agg * dis_ref[...] + b2_ref[...], 0.0)
        out_ref[...] = (jnp.dot(z[:N], wfc_ref[...],
                                preferred_element_type=f32) + bfc_ref[...])

    return pl.pallas_call(
        body,
        out_shape=jax.ShapeDtypeStruct((N, 112), f32),
    )(s2, tbl2, dis, b2, Wfc, bfc)


def kernel(x, edge_index, W1, b1, W2, b2, Wfc, bfc):
    ei = edge_index.astype(jnp.int32)
    src, dst = ei[0], ei[1]
    degp = _deg(dst)
    h = _tc_mm(x, W1)
    tbl1, dis = _tc_a(h, degp.reshape(NC, NPAD, 1))
    s1 = _agg128(tbl1, src, dst)
    tbl2 = _tc_b(s1, tbl1, dis, b1, W2)
    s2 = _agg64(tbl2, src, dst)
    return _tc_c(s2, tbl2, dis, b2, Wfc, bfc)


# speedup vs baseline: 36.4845x; 1.0641x over previous
import functools

import jax
import jax.numpy as jnp
from jax import lax
from jax.experimental import pallas as pl
from jax.experimental.pallas import tpu as pltpu
from jax.experimental.pallas import tpu_sc as plsc

N = 10000
NPAD = 10240
E = 320000
NC, NS = 2, 16
NW = NC * NS
EPW = E // NW
K = 80
RPT = NPAD // NS

f32 = jnp.float32
_MESH = functools.partial(
    plsc.VectorSubcoreMesh, core_axis_name="c", subcore_axis_name="s")


NCHUNK = EPW // K


DNBUF = 5
DNGRP = NCHUNK // DNBUF


def _deg_body(ei_hbm, out_hbm, ones_v, zbuf_v, acc_sh, *bufs):
    idx_v = bufs[:DNBUF]
    sem_i = bufs[DNBUF:2 * DNBUF]
    sem_s = bufs[2 * DNBUF:3 * DNBUF]
    c = lax.axis_index("c")
    s = lax.axis_index("s")
    for j in range(K // 16):
        ones_v[pl.ds(j * 16, 16)] = jnp.full((16,), 1.0, f32)
    for j in range(RPT // 16):
        zbuf_v[pl.ds(j * 16, 16)] = jnp.zeros((16,), f32)
    pltpu.sync_copy(zbuf_v, acc_sh.at[pl.ds(s * RPT, RPT)])
    plsc.subcore_barrier()
    base = E + (c * NS + s) * EPW

    for b in range(DNBUF):
        pltpu.async_copy(ei_hbm.at[pl.ds(base + b * K, K)], idx_v[b],
                         sem_i[b])

    def group(g, carry):
        scs = []
        for b in range(DNBUF):
            off = base + (g * DNBUF + b) * K
            pltpu.make_async_copy(ei_hbm.at[pl.ds(off, K)], idx_v[b],
                                  sem_i[b]).wait()
            scs.append(pltpu.async_copy(ones_v, acc_sh.at[idx_v[b]],
                                        sem_s[b], add=True))
        for b in range(DNBUF):
            scs[b].wait()

            @pl.when(g < DNGRP - 1)
            def _():
                off2 = base + ((g + 1) * DNBUF + b) * K
                pltpu.async_copy(ei_hbm.at[pl.ds(off2, K)], idx_v[b],
                                 sem_i[b])
        return carry

    lax.fori_loop(0, DNGRP, group, 0)
    plsc.subcore_barrier()
    pltpu.sync_copy(acc_sh.at[pl.ds(s * RPT, RPT)],
                    out_hbm.at[c].at[pl.ds(s * RPT, RPT)])


def _deg(ei):
    return pl.kernel(
        _deg_body,
        mesh=_MESH(),
        out_type=jax.ShapeDtypeStruct((NC, NPAD), f32),
        scratch_types=(
            [pltpu.VMEM((K,), f32),
             pltpu.VMEM((RPT,), f32),
             pltpu.VMEM_SHARED((NPAD,), f32)]
            + [pltpu.VMEM((K,), jnp.int32) for _ in range(DNBUF)]
            + [pltpu.SemaphoreType.DMA for _ in range(2 * DNBUF)]
        ),
    )(ei)


AK = 40
ANC = EPW // AK
RBUF = 5
IBUF = 2 * RBUF
ANIT = ANC // IBUF


def _make_agg(D, tc_tiling):
    def body(tbl_hbm, ei_hbm, out_hbm, acc_sh, *bufs):
        idx_s = bufs[0 * IBUF:1 * IBUF]
        idx_d = bufs[1 * IBUF:2 * IBUF]
        rows = bufs[2 * IBUF:2 * IBUF + RBUF]
        o = 2 * IBUF + RBUF
        sem_is = bufs[o:o + IBUF]
        sem_id = bufs[o + IBUF:o + 2 * IBUF]
        sem_g = bufs[o + 2 * IBUF:o + 2 * IBUF + RBUF]
        sem_sc = bufs[o + 2 * IBUF + RBUF:o + 2 * IBUF + 2 * RBUF]
        c = lax.axis_index("c")
        s = lax.axis_index("s")
        base = (c * NS + s) * EPW

        def start_idx(i, slot):
            off = base + i * AK
            pltpu.async_copy(ei_hbm.at[pl.ds(off, AK)], idx_s[slot],
                             sem_is[slot])
            pltpu.async_copy(ei_hbm.at[pl.ds(E + off, AK)], idx_d[slot],
                             sem_id[slot])

        def wait_is(i, slot):
            pltpu.make_async_copy(ei_hbm.at[pl.ds(base + i * AK, AK)],
                                  idx_s[slot], sem_is[slot]).wait()

        def wait_id(i, slot):
            pltpu.make_async_copy(ei_hbm.at[pl.ds(E + base + i * AK, AK)],
                                  idx_d[slot], sem_id[slot]).wait()

        def start_gather(slot, r):
            return pltpu.async_copy(tbl_hbm.at[idx_s[slot]], rows[r],
                                    sem_g[r])

        def wait_gather(slot, r):
            pltpu.make_async_copy(tbl_hbm.at[idx_s[slot]], rows[r],
                                  sem_g[r]).wait()

        def start_scatter(slot, r):
            return pltpu.async_copy(rows[r], acc_sh.at[idx_d[slot]],
                                    sem_sc[r], add=True)

        def wait_scatter(slot, r):
            pltpu.make_async_copy(rows[r], acc_sh.at[idx_d[slot]],
                                  sem_sc[r]).wait()

        for j in range(RBUF):
            start_idx(j, j)
        pltpu.sync_copy(tbl_hbm.at[pl.ds(s * RPT, RPT)],
                        acc_sh.at[pl.ds(s * RPT, RPT)])
        plsc.subcore_barrier()

        def iteration(t, carry):
            i0 = t * IBUF
            for j in range(IBUF):
                i = i0 + j
                r = j % RBUF
                jm2, rm2 = (j - 2) % IBUF, (j - 2) % RBUF
                jm5 = (j - RBUF) % IBUF

                def step1():
                    wait_scatter(jm5, r)

                if j >= RBUF:
                    step1()
                else:
                    pl.when(t > 0)(step1)
                wait_is(i, j)
                start_gather(j, r)

                def step3():
                    wait_gather(jm2, rm2)
                    wait_id(i - 2, jm2)
                    start_scatter(jm2, rm2)

                if j >= 2:
                    step3()
                else:
                    pl.when(t > 0)(step3)

                def step4():
                    start_idx(i + RBUF, (j + RBUF) % IBUF)

                if j < RBUF:
                    step4()
                else:
                    pl.when(t < ANIT - 1)(step4)
            return carry

        lax.fori_loop(0, ANIT, iteration, 0)
        last = ANC - 2
        for i in range(last, ANC):
            j, r = i % IBUF, i % RBUF
            wait_gather(j, r)
            wait_id(i, j)
            start_scatter(j, r)
        for i in range(ANC - RBUF, ANC):
            wait_scatter(i % IBUF, i % RBUF)
        plsc.subcore_barrier()
        pltpu.sync_copy(acc_sh.at[pl.ds(s * RPT, RPT)],
                        out_hbm.at[c].at[pl.ds(s * RPT, RPT)])

    def run(tbl, ei):
        return pl.kernel(
            body,
            mesh=_MESH(),
            out_type=jax.ShapeDtypeStruct((NC, NPAD, D), f32),
            scratch_types=(
                [pltpu.VMEM_SHARED((NPAD, D), f32)]
                + [pltpu.VMEM((AK,), jnp.int32) for _ in range(2 * IBUF)]
                + [pltpu.VMEM((AK, D), f32) for _ in range(RBUF)]
                + [pltpu.SemaphoreType.DMA for _ in range(2 * IBUF + 2 * RBUF)]
            ),
            compiler_params=pltpu.CompilerParams(
                use_tc_tiling_on_sc=tc_tiling),
        )(tbl, ei)

    return run


_agg128 = _make_agg(128, True)
_agg64 = _make_agg(64, False)


BRA = 1024
BRC = 1000


def _tc_mm(x, W1):
    def body(x_ref, w_ref, h_ref):
        h_ref[...] = jnp.dot(x_ref[...], w_ref[...],
                             preferred_element_type=f32)

    return pl.pallas_call(
        body,
        grid=(5,),
        in_specs=[pl.BlockSpec((2000, 128), lambda i: (i, 0)),
                  pl.BlockSpec((128, 128), lambda i: (0, 0))],
        out_specs=pl.BlockSpec((2000, 128), lambda i: (i, 0)),
        out_shape=jax.ShapeDtypeStruct((N, 128), f32),
    )(x, W1)


def _row_mask(i, br):
    rows = i * br + lax.broadcasted_iota(jnp.int32, (br, 1), 0)
    return rows < N


def _tc_a(h, degp):
    def body(h_ref, degp_ref, tbl_ref, dis_ref):
        i = pl.program_id(0)
        deg = degp_ref[0:1] + degp_ref[1:2] + 1.0
        dis = lax.rsqrt(deg)
        dis_ref[...] = dis[0]
        disc = jnp.reshape(dis, (BRA, 1))
        tbl_ref[...] = jnp.where(_row_mask(i, BRA),
                                 h_ref[...] * disc, 0.0)

    return pl.pallas_call(
        body,
        grid=(NPAD // BRA,),
        in_specs=[pl.BlockSpec((BRA, 128), lambda i: (i, 0)),
                  pl.BlockSpec((2, BRA), lambda i: (0, i))],
        out_specs=(pl.BlockSpec((BRA, 128), lambda i: (i, 0)),
                   pl.BlockSpec((BRA,), lambda i: (i,))),
        out_shape=(jax.ShapeDtypeStruct((NPAD, 128), f32),
                   jax.ShapeDtypeStruct((NPAD,), f32)),
    )(h, degp)


def _tc_b(s1, tbl1, dis, b1, W2):
    def body(s1_ref, tbl_ref, dis_ref, b1_ref, w2_ref, out_ref):
        i = pl.program_id(0)
        disc = jnp.reshape(dis_ref[...], (BRA, 1))
        agg = s1_ref[0] + s1_ref[1] - tbl_ref[...]
        z = jnp.maximum(agg * disc + b1_ref[...], 0.0)
        z = jnp.where(_row_mask(i, BRA), z, 0.0)
        h2 = jnp.dot(z, w2_ref[...], preferred_element_type=f32)
        out_ref[...] = h2 * disc

    return pl.pallas_call(
        body,
        grid=(NPAD // BRA,),
        in_specs=[pl.BlockSpec((2, BRA, 128), lambda i: (0, i, 0)),
                  pl.BlockSpec((BRA, 128), lambda i: (i, 0)),
                  pl.BlockSpec((BRA,), lambda i: (i,)),
                  pl.BlockSpec((128,), lambda i: (0,)),
                  pl.BlockSpec((128, 64), lambda i: (0, 0))],
        out_specs=pl.BlockSpec((BRA, 64), lambda i: (i, 0)),
        out_shape=jax.ShapeDtypeStruct((NPAD, 64), f32),
    )(s1, tbl1, dis, b1, W2)


def _tc_c(s2, tbl2, dis, b2, Wfc, bfc):
    def body(s2_ref, tbl_ref, dis_ref, b2_ref, wfc_ref, bfc_ref, out_ref):
        i = pl.program_id(0)
        disc = jnp.reshape(dis_ref[...], (BRA, 1))
        agg = s2_ref[0] + s2_ref[1] - tbl_ref[...]
        z = jnp.maximum(agg * disc + b2_ref[...], 0.0)
        z = jnp.where(_row_mask(i, BRA), z, 0.0)
        out_ref[...] = (jnp.dot(z, wfc_ref[...],
                                preferred_element_type=f32) + bfc_ref[...])

    return pl.pallas_call(
        body,
        grid=(NPAD // BRA,),
        in_specs=[pl.BlockSpec((2, BRA, 64), lambda i: (0, i, 0)),
                  pl.BlockSpec((BRA, 64), lambda i: (i, 0)),
                  pl.BlockSpec((BRA,), lambda i: (i,)),
                  pl.BlockSpec((64,), lambda i: (0,)),
                  pl.BlockSpec((64, 112), lambda i: (0, 0)),
                  pl.BlockSpec((112,), lambda i: (0,))],
        out_specs=pl.BlockSpec((BRA, 112), lambda i: (i, 0)),
        out_shape=jax.ShapeDtypeStruct((N, 112), f32),
    )(s2, tbl2, dis, b2, Wfc, bfc)


def kernel(x, edge_index, W1, b1, W2, b2, Wfc, bfc):
    ei = edge_index.astype(jnp.int32).reshape(2 * E)
    degp = _deg(ei)
    h = _tc_mm(x, W1)
    tbl1, dis = _tc_a(h, degp)
    s1 = _agg128(tbl1, ei)
    tbl2 = _tc_b(s1, tbl1, dis, b1, W2)
    s2 = _agg64(tbl2, ei)
    return _tc_c(s2, tbl2, dis, b2, Wfc, bfc)
